# Initial kernel scaffold; baseline (speedup 1.0000x reference)
import jax
import jax.numpy as jnp
from jax.experimental import pallas as pl


def kernel(feature, jmap, joff, fc1_w, fc1_b, w1, b1, w2, b2, w3, b3):
    raise NotImplementedError("write your pallas kernel here")


# speedup vs baseline: 14.6158x; 14.6158x over previous
import functools

import jax
import jax.numpy as jnp
from jax import lax
from jax.experimental import pallas as pl
from jax.experimental.pallas import tpu as pltpu
from jax.experimental.pallas import tpu_sc as plsc

N_PTS0 = 32
N_PTS1 = 8
DIM_LOI = 128
DIM_FC = 1024
K = 64
H = 128
W = 128
C_FEAT = 256
NPIX = H * W
NLINES = K * K
NPTS = NLINES * N_PTS0

SC_NC = 2
SC_NS = 16
SC_NW = SC_NC * SC_NS
LINES_PER_W = NLINES // SC_NW
PTS_PER_W = LINES_PER_W * N_PTS0
G_LINES = 2
G_PTS = G_LINES * N_PTS0
N_GROUPS = LINES_PER_W // G_LINES


def _fc1_body(f_ref, w_ref, b_ref, o_ref):
    o_ref[...] = lax.dot_general(
        f_ref[...], w_ref[...], (((0,), (1,)), ((), ())),
        preferred_element_type=jnp.float32) + b_ref[...]


def _fc1_call(feat2d, fc1_w, fc1_b_row):
    blk = 2048
    grid = NPIX // blk
    return pl.pallas_call(
        _fc1_body,
        grid=(grid,),
        in_specs=[
            pl.BlockSpec((C_FEAT, blk), lambda i: (0, i)),
            pl.BlockSpec((DIM_LOI, C_FEAT), lambda i: (0, 0)),
            pl.BlockSpec((1, DIM_LOI), lambda i: (0, 0)),
        ],
        out_specs=pl.BlockSpec((blk, DIM_LOI), lambda i: (i, 0)),
        out_shape=jax.ShapeDtypeStruct((NPIX, DIM_LOI), jnp.float32),
    )(feat2d, fc1_w, fc1_b_row)


def _junction_body(jmap_ref, joff0_ref, joff1_ref,
                   i00_ref, i10_ref, i01_ref, i11_ref,
                   w00_ref, w10_ref, w01_ref, w11_ref):
    a = jmap_ref[...]
    neg = jnp.float32(-jnp.inf)
    negrow = jnp.full((1, W), neg, jnp.float32)
    up = jnp.concatenate([a[1:, :], negrow], axis=0)
    dn = jnp.concatenate([negrow, a[:-1, :]], axis=0)
    v = jnp.maximum(a, jnp.maximum(up, dn))
    negcol = jnp.full((H, 1), neg, jnp.float32)
    lf = jnp.concatenate([v[:, 1:], negcol], axis=1)
    rt = jnp.concatenate([negcol, v[:, :-1]], axis=1)
    ap = jnp.maximum(v, jnp.maximum(lf, rt))
    jm = a * (a == ap).astype(jnp.float32)

    joff0 = joff0_ref[...]
    joff1 = joff1_ref[...]
    ri = lax.broadcasted_iota(jnp.int32, (H, W), 0)
    ci = lax.broadcasted_iota(jnp.int32, (H, W), 1)
    flatid = ri * W + ci

    kcol = lax.broadcasted_iota(jnp.int32, (K, 1), 0)
    qrow = lax.broadcasted_iota(jnp.int32, (1, K * N_PTS0), 1)
    vrow = qrow // N_PTS0

    def step(k, carry):
        jm_c, ycol, xcol, yrow, xrow = carry
        m = jnp.max(jm_c)
        sel = jm_c == m
        idx = jnp.min(jnp.where(sel, flatid, jnp.int32(1 << 30)))
        onehot = flatid == idx
        jy = jnp.sum(jnp.where(onehot, joff0, 0.0))
        jx = jnp.sum(jnp.where(onehot, joff1, 0.0))
        yk = (idx // W).astype(jnp.float32) + jy + 0.5
        xk = (idx % W).astype(jnp.float32) + jx + 0.5
        jm_c = jnp.where(onehot, neg, jm_c)
        ycol = jnp.where(kcol == k, yk, ycol)
        xcol = jnp.where(kcol == k, xk, xcol)
        yrow = jnp.where(vrow == k, yk, yrow)
        xrow = jnp.where(vrow == k, xk, xrow)
        return jm_c, ycol, xcol, yrow, xrow

    z_col = jnp.zeros((K, 1), jnp.float32)
    z_row = jnp.zeros((1, K * N_PTS0), jnp.float32)
    _, ycol, xcol, yrow, xrow = lax.fori_loop(
        0, K, step, (jm, z_col, z_col, z_row, z_row))

    t = (qrow % N_PTS0).astype(jnp.float32)
    lam = t / jnp.float32(N_PTS0 - 1)
    px = ycol * lam + yrow * (1.0 - lam) - 0.5
    py = xcol * lam + xrow * (1.0 - lam) - 0.5
    px0 = jnp.clip(jnp.floor(px), 0.0, H - 1.0)
    py0 = jnp.clip(jnp.floor(py), 0.0, W - 1.0)
    px1 = jnp.clip(px0 + 1.0, 0.0, H - 1.0)
    py1 = jnp.clip(py0 + 1.0, 0.0, W - 1.0)
    px0i = px0.astype(jnp.int32)
    py0i = py0.astype(jnp.int32)
    px1i = px1.astype(jnp.int32)
    py1i = py1.astype(jnp.int32)
    i00_ref[...] = px0i * W + py0i
    i10_ref[...] = px1i * W + py0i
    i01_ref[...] = px0i * W + py1i
    i11_ref[...] = px1i * W + py1i
    w00_ref[...] = (px1 - px) * (py1 - py)
    w10_ref[...] = (px - px0) * (py1 - py)
    w01_ref[...] = (px1 - px) * (py - py0)
    w11_ref[...] = (px - px0) * (py - py0)


def _junction_call(jmap2d, joff0, joff1):
    shp = jax.ShapeDtypeStruct((K, K * N_PTS0), jnp.int32)
    shpf = jax.ShapeDtypeStruct((K, K * N_PTS0), jnp.float32)
    return pl.pallas_call(
        _junction_body,
        out_shape=(shp, shp, shp, shp, shpf, shpf, shpf, shpf),
    )(jmap2d, joff0, joff1)


def _sc_gather_kernel(table, i00, i10, i01, i11, w00, w10, w01, w11):
    mesh = plsc.VectorSubcoreMesh(core_axis_name="c", subcore_axis_name="s")

    @functools.partial(
        pl.kernel, mesh=mesh,
        out_type=jax.ShapeDtypeStruct((NLINES * DIM_LOI * N_PTS1,), jnp.float32),
        scratch_types=[
            pltpu.VMEM((PTS_PER_W,), jnp.int32),
            pltpu.VMEM((PTS_PER_W,), jnp.int32),
            pltpu.VMEM((PTS_PER_W,), jnp.int32),
            pltpu.VMEM((PTS_PER_W,), jnp.int32),
            pltpu.VMEM((PTS_PER_W,), jnp.float32),
            pltpu.VMEM((PTS_PER_W,), jnp.float32),
            pltpu.VMEM((PTS_PER_W,), jnp.float32),
            pltpu.VMEM((PTS_PER_W,), jnp.float32),
            pltpu.VMEM((G_PTS, DIM_LOI), jnp.float32),
            pltpu.VMEM((G_PTS, DIM_LOI), jnp.float32),
            pltpu.VMEM((G_PTS, DIM_LOI), jnp.float32),
            pltpu.VMEM((G_PTS, DIM_LOI), jnp.float32),
            pltpu.VMEM((G_LINES * DIM_LOI * N_PTS1,), jnp.float32),
            pltpu.SemaphoreType.DMA,
        ],
    )
    def k(table_h, i00_h, i10_h, i01_h, i11_h, w00_h, w10_h, w01_h, w11_h,
          out_h, i00_v, i10_v, i01_v, i11_v, w00_v, w10_v, w01_v, w11_v,
          r00, r10, r01, r11, obuf, sem):
        wid = lax.axis_index("s") * SC_NC + lax.axis_index("c")
        pbase = wid * PTS_PER_W
        pltpu.sync_copy(i00_h.at[pl.ds(pbase, PTS_PER_W)], i00_v)
        pltpu.sync_copy(i10_h.at[pl.ds(pbase, PTS_PER_W)], i10_v)
        pltpu.sync_copy(i01_h.at[pl.ds(pbase, PTS_PER_W)], i01_v)
        pltpu.sync_copy(i11_h.at[pl.ds(pbase, PTS_PER_W)], i11_v)
        pltpu.sync_copy(w00_h.at[pl.ds(pbase, PTS_PER_W)], w00_v)
        pltpu.sync_copy(w10_h.at[pl.ds(pbase, PTS_PER_W)], w10_v)
        pltpu.sync_copy(w01_h.at[pl.ds(pbase, PTS_PER_W)], w01_v)
        pltpu.sync_copy(w11_h.at[pl.ds(pbase, PTS_PER_W)], w11_v)

        def group(g, _):
            goff = g * G_PTS
            c0 = pltpu.async_copy(table_h.at[i00_v.at[pl.ds(goff, G_PTS)]], r00, sem)
            c1 = pltpu.async_copy(table_h.at[i10_v.at[pl.ds(goff, G_PTS)]], r10, sem)
            c2 = pltpu.async_copy(table_h.at[i01_v.at[pl.ds(goff, G_PTS)]], r01, sem)
            c3 = pltpu.async_copy(table_h.at[i11_v.at[pl.ds(goff, G_PTS)]], r11, sem)
            c0.wait(); c1.wait(); c2.wait(); c3.wait()

            def chunk(pg, _):
                base = pg * 16
                w00c = w00_v[pl.ds(goff + base, 16)]
                w10c = w10_v[pl.ds(goff + base, 16)]
                w01c = w01_v[pl.ds(goff + base, 16)]
                w11c = w11_v[pl.ds(goff + base, 16)]
                for b16 in range(16):
                    b = base + b16
                    ii = jnp.full((16,), b16, jnp.int32)
                    s00 = jnp.take_along_axis(w00c, ii, axis=0,
                                              mode="promise_in_bounds")
                    s10 = jnp.take_along_axis(w10c, ii, axis=0,
                                              mode="promise_in_bounds")
                    s01 = jnp.take_along_axis(w01c, ii, axis=0,
                                              mode="promise_in_bounds")
                    s11 = jnp.take_along_axis(w11c, ii, axis=0,
                                              mode="promise_in_bounds")
                    p_loc = pg * 4 + (b16 // 4)
                    q = b16 % 4
                    for j in range(DIM_LOI // 16):
                        acc = (r00[b, pl.ds(16 * j, 16)] * s00
                               + r10[b, pl.ds(16 * j, 16)] * s10
                               + r01[b, pl.ds(16 * j, 16)] * s01
                               + r11[b, pl.ds(16 * j, 16)] * s11)
                        off = p_loc * DIM_LOI + 16 * j
                        if q == 0:
                            obuf[pl.ds(off, 16)] = acc
                        else:
                            obuf[pl.ds(off, 16)] = jnp.maximum(
                                obuf[pl.ds(off, 16)], acc)
                return 0

            lax.fori_loop(0, G_PTS // 16, chunk, 0)
            obase = (wid * LINES_PER_W + g * G_LINES) * DIM_LOI * N_PTS1
            pltpu.sync_copy(obuf, out_h.at[pl.ds(obase, G_LINES * DIM_LOI * N_PTS1)])
            return 0

        lax.fori_loop(0, N_GROUPS, group, 0)

    return k(table, i00, i10, i01, i11, w00, w10, w01, w11)


def _mlp_body(x_ref, w1_ref, b1_ref, w2_ref, b2_ref, w3_ref, b3_ref, o_ref):
    dn = (((1,), (0,)), ((), ()))
    h1 = jax.nn.relu(lax.dot_general(x_ref[...], w1_ref[...], dn,
                                     preferred_element_type=jnp.float32)
                     + b1_ref[...])
    h2 = jax.nn.relu(lax.dot_general(h1, w2_ref[...], dn,
                                     preferred_element_type=jnp.float32)
                     + b2_ref[...])
    lg = lax.dot_general(h2, w3_ref[...], dn,
                         preferred_element_type=jnp.float32) + b3_ref[...]
    m = jnp.max(lg, axis=1, keepdims=True)
    e = jnp.exp(lg - m)
    o_ref[...] = e / jnp.sum(e, axis=1, keepdims=True)


def _mlp_call(feat, w1p, b1_row, w2, b2_row, w3p, b3p_row):
    blk = 1024
    grid = NLINES // blk
    return pl.pallas_call(
        _mlp_body,
        grid=(grid,),
        in_specs=[
            pl.BlockSpec((blk, DIM_FC), lambda i: (i, 0)),
            pl.BlockSpec((DIM_FC, DIM_FC), lambda i: (0, 0)),
            pl.BlockSpec((1, DIM_FC), lambda i: (0, 0)),
            pl.BlockSpec((DIM_FC, DIM_FC), lambda i: (0, 0)),
            pl.BlockSpec((1, DIM_FC), lambda i: (0, 0)),
            pl.BlockSpec((DIM_FC, 128), lambda i: (0, 0)),
            pl.BlockSpec((1, 128), lambda i: (0, 0)),
        ],
        out_specs=pl.BlockSpec((blk, 128), lambda i: (i, 0)),
        out_shape=jax.ShapeDtypeStruct((NLINES, 128), jnp.float32),
    )(feat, w1p, b1_row, w2, b2_row, w3p, b3p_row)


def kernel(feature, jmap, joff, fc1_w, fc1_b, w1, b1, w2, b2, w3, b3):
    feat2d = feature.reshape(C_FEAT, NPIX)
    jmap2d = jmap.reshape(H, W)
    joff0 = joff[0, 0, 0]
    joff1 = joff[0, 0, 1]

    table = _fc1_call(feat2d, fc1_w, fc1_b.reshape(1, DIM_LOI))

    i00, i10, i01, i11, w00, w10, w01, w11 = _junction_call(
        jmap2d, joff0, joff1)

    feat_lines = _sc_gather_kernel(
        table,
        i00.reshape(NPTS), i10.reshape(NPTS),
        i01.reshape(NPTS), i11.reshape(NPTS),
        w00.reshape(NPTS), w10.reshape(NPTS),
        w01.reshape(NPTS), w11.reshape(NPTS),
    ).reshape(NLINES, DIM_FC)

    w1p = w1.reshape(DIM_LOI, N_PTS1, DIM_FC).transpose(1, 0, 2).reshape(
        DIM_FC, DIM_FC)
    w3p = jnp.pad(w3, ((0, 0), (0, 125)))
    b3p = jnp.concatenate([b3, jnp.full((125,), -1e30, jnp.float32)])

    probs = _mlp_call(feat_lines, w1p, b1.reshape(1, DIM_FC),
                      w2, b2.reshape(1, DIM_FC), w3p, b3p.reshape(1, 128))
    return probs[:, :3]


# speedup vs baseline: 28.0396x; 1.9184x over previous
import functools

import jax
import jax.numpy as jnp
from jax import lax
from jax.experimental import pallas as pl
from jax.experimental.pallas import tpu as pltpu
from jax.experimental.pallas import tpu_sc as plsc

N_PTS0 = 32
N_PTS1 = 8
DIM_LOI = 128
DIM_FC = 1024
K = 64
H = 128
W = 128
C_FEAT = 256
NPIX = H * W
NLINES = K * K
NPTS = NLINES * N_PTS0

SC_NC = 2
SC_NS = 16
SC_NW = SC_NC * SC_NS
LINES_PER_W = NLINES // SC_NW
PTS_PER_W = LINES_PER_W * N_PTS0
G_LINES = 2
G_PTS = G_LINES * N_PTS0
N_GROUPS = LINES_PER_W // G_LINES


def _fc1_body(f_ref, w_ref, b_ref, o_ref):
    o_ref[...] = lax.dot_general(
        f_ref[...], w_ref[...], (((0,), (1,)), ((), ())),
        preferred_element_type=jnp.float32) + b_ref[...]


def _fc1_call(feat2d, fc1_w, fc1_b_row):
    blk = 2048
    grid = NPIX // blk
    return pl.pallas_call(
        _fc1_body,
        grid=(grid,),
        in_specs=[
            pl.BlockSpec((C_FEAT, blk), lambda i: (0, i)),
            pl.BlockSpec((DIM_LOI, C_FEAT), lambda i: (0, 0)),
            pl.BlockSpec((1, DIM_LOI), lambda i: (0, 0)),
        ],
        out_specs=pl.BlockSpec((blk, DIM_LOI), lambda i: (i, 0)),
        out_shape=jax.ShapeDtypeStruct((NPIX, DIM_LOI), jnp.float32),
    )(feat2d, fc1_w, fc1_b_row)


def _junction_body(jmap_ref, joff0_ref, joff1_ref,
                   i00_ref, i10_ref, i01_ref, i11_ref,
                   w00_ref, w10_ref, w01_ref, w11_ref):
    a = jmap_ref[...]
    neg = jnp.float32(-jnp.inf)
    negrow = jnp.full((1, W), neg, jnp.float32)
    up = jnp.concatenate([a[1:, :], negrow], axis=0)
    dn = jnp.concatenate([negrow, a[:-1, :]], axis=0)
    v = jnp.maximum(a, jnp.maximum(up, dn))
    negcol = jnp.full((H, 1), neg, jnp.float32)
    lf = jnp.concatenate([v[:, 1:], negcol], axis=1)
    rt = jnp.concatenate([negcol, v[:, :-1]], axis=1)
    ap = jnp.maximum(v, jnp.maximum(lf, rt))
    jm = a * (a == ap).astype(jnp.float32)

    joff0 = joff0_ref[...]
    joff1 = joff1_ref[...]
    ri = lax.broadcasted_iota(jnp.int32, (H, W), 0)
    ci = lax.broadcasted_iota(jnp.int32, (H, W), 1)
    flatid = ri * W + ci

    kcol = lax.broadcasted_iota(jnp.int32, (K, 1), 0)
    qrow = lax.broadcasted_iota(jnp.int32, (1, K * N_PTS0), 1)
    vrow = qrow // N_PTS0

    def step(k, carry):
        jm_c, ycol, xcol, yrow, xrow = carry
        m = jnp.max(jm_c)
        sel = jm_c == m
        idx = jnp.min(jnp.where(sel, flatid, jnp.int32(1 << 30)))
        onehot = flatid == idx
        jy = jnp.sum(jnp.where(onehot, joff0, 0.0))
        jx = jnp.sum(jnp.where(onehot, joff1, 0.0))
        yk = (idx // W).astype(jnp.float32) + jy + 0.5
        xk = (idx % W).astype(jnp.float32) + jx + 0.5
        jm_c = jnp.where(onehot, neg, jm_c)
        ycol = jnp.where(kcol == k, yk, ycol)
        xcol = jnp.where(kcol == k, xk, xcol)
        yrow = jnp.where(vrow == k, yk, yrow)
        xrow = jnp.where(vrow == k, xk, xrow)
        return jm_c, ycol, xcol, yrow, xrow

    z_col = jnp.zeros((K, 1), jnp.float32)
    z_row = jnp.zeros((1, K * N_PTS0), jnp.float32)
    _, ycol, xcol, yrow, xrow = lax.fori_loop(
        0, K, step, (jm, z_col, z_col, z_row, z_row))

    t = (qrow % N_PTS0).astype(jnp.float32)
    lam = t / jnp.float32(N_PTS0 - 1)
    px = ycol * lam + yrow * (1.0 - lam) - 0.5
    py = xcol * lam + xrow * (1.0 - lam) - 0.5
    px0 = jnp.clip(jnp.floor(px), 0.0, H - 1.0)
    py0 = jnp.clip(jnp.floor(py), 0.0, W - 1.0)
    px1 = jnp.clip(px0 + 1.0, 0.0, H - 1.0)
    py1 = jnp.clip(py0 + 1.0, 0.0, W - 1.0)
    px0i = px0.astype(jnp.int32)
    py0i = py0.astype(jnp.int32)
    px1i = px1.astype(jnp.int32)
    py1i = py1.astype(jnp.int32)
    i00_ref[...] = px0i * W + py0i
    i10_ref[...] = px1i * W + py0i
    i01_ref[...] = px0i * W + py1i
    i11_ref[...] = px1i * W + py1i
    w00_ref[...] = (px1 - px) * (py1 - py)
    w10_ref[...] = (px - px0) * (py1 - py)
    w01_ref[...] = (px1 - px) * (py - py0)
    w11_ref[...] = (px - px0) * (py - py0)


def _junction_call(jmap2d, joff0, joff1):
    shp = jax.ShapeDtypeStruct((K, K * N_PTS0), jnp.int32)
    shpf = jax.ShapeDtypeStruct((K, K * N_PTS0), jnp.float32)
    return pl.pallas_call(
        _junction_body,
        out_shape=(shp, shp, shp, shp, shpf, shpf, shpf, shpf),
    )(jmap2d, joff0, joff1)


def _sc_gather_kernel(table, i00, i10, i01, i11, w00, w10, w01, w11):
    mesh = plsc.VectorSubcoreMesh(core_axis_name="c", subcore_axis_name="s")

    rbuf_t = pltpu.VMEM((G_PTS, DIM_LOI), jnp.float32)

    @functools.partial(
        pl.kernel, mesh=mesh,
        out_type=jax.ShapeDtypeStruct((NLINES * DIM_LOI * N_PTS1,), jnp.float32),
        scratch_types=[
            pltpu.VMEM((PTS_PER_W,), jnp.int32),
            pltpu.VMEM((PTS_PER_W,), jnp.int32),
            pltpu.VMEM((PTS_PER_W,), jnp.int32),
            pltpu.VMEM((PTS_PER_W,), jnp.int32),
            pltpu.VMEM((PTS_PER_W,), jnp.float32),
            pltpu.VMEM((PTS_PER_W,), jnp.float32),
            pltpu.VMEM((PTS_PER_W,), jnp.float32),
            pltpu.VMEM((PTS_PER_W,), jnp.float32),
            rbuf_t, rbuf_t, rbuf_t, rbuf_t,
            rbuf_t, rbuf_t, rbuf_t, rbuf_t,
            pltpu.VMEM((G_LINES * DIM_LOI * N_PTS1,), jnp.float32),
            pltpu.SemaphoreType.DMA,
            pltpu.SemaphoreType.DMA,
        ],
    )
    def k(table_h, i00_h, i10_h, i01_h, i11_h, w00_h, w10_h, w01_h, w11_h,
          out_h, i00_v, i10_v, i01_v, i11_v, w00_v, w10_v, w01_v, w11_v,
          a00, a10, a01, a11, b00, b10, b01, b11, obuf, semA, semB):
        wid = lax.axis_index("s") * SC_NC + lax.axis_index("c")
        pbase = wid * PTS_PER_W
        pltpu.sync_copy(i00_h.at[pl.ds(pbase, PTS_PER_W)], i00_v)
        pltpu.sync_copy(i10_h.at[pl.ds(pbase, PTS_PER_W)], i10_v)
        pltpu.sync_copy(i01_h.at[pl.ds(pbase, PTS_PER_W)], i01_v)
        pltpu.sync_copy(i11_h.at[pl.ds(pbase, PTS_PER_W)], i11_v)
        pltpu.sync_copy(w00_h.at[pl.ds(pbase, PTS_PER_W)], w00_v)
        pltpu.sync_copy(w10_h.at[pl.ds(pbase, PTS_PER_W)], w10_v)
        pltpu.sync_copy(w01_h.at[pl.ds(pbase, PTS_PER_W)], w01_v)
        pltpu.sync_copy(w11_h.at[pl.ds(pbase, PTS_PER_W)], w11_v)

        ivs = (i00_v, i10_v, i01_v, i11_v)

        def fire(g, bufs, sem):
            goff = g * G_PTS
            for iv, rb in zip(ivs, bufs):
                pltpu.async_copy(table_h.at[iv.at[pl.ds(goff, G_PTS)]], rb, sem)

        def drain(bufs, sem):
            for iv, rb in zip(ivs, bufs):
                pltpu.make_async_copy(
                    table_h.at[iv.at[pl.ds(0, G_PTS)]], rb, sem).wait()

        def compute(g, bufs):
            r00, r10, r01, r11 = bufs
            goff = g * G_PTS

            def chunk(pg, _):
                base = pg * 16
                w00c = w00_v[pl.ds(goff + base, 16)]
                w10c = w10_v[pl.ds(goff + base, 16)]
                w01c = w01_v[pl.ds(goff + base, 16)]
                w11c = w11_v[pl.ds(goff + base, 16)]
                for pp in range(4):
                    accs = [None] * (DIM_LOI // 16)
                    for q in range(4):
                        b16 = pp * 4 + q
                        b = base + b16
                        ii = jnp.full((16,), b16, jnp.int32)
                        s00 = jnp.take_along_axis(w00c, ii, axis=0,
                                                  mode="promise_in_bounds")
                        s10 = jnp.take_along_axis(w10c, ii, axis=0,
                                                  mode="promise_in_bounds")
                        s01 = jnp.take_along_axis(w01c, ii, axis=0,
                                                  mode="promise_in_bounds")
                        s11 = jnp.take_along_axis(w11c, ii, axis=0,
                                                  mode="promise_in_bounds")
                        for j in range(DIM_LOI // 16):
                            acc = (r00[b, pl.ds(16 * j, 16)] * s00
                                   + r10[b, pl.ds(16 * j, 16)] * s10
                                   + r01[b, pl.ds(16 * j, 16)] * s01
                                   + r11[b, pl.ds(16 * j, 16)] * s11)
                            accs[j] = acc if q == 0 else jnp.maximum(accs[j],
                                                                     acc)
                    p_loc = pg * 4 + pp
                    for j in range(DIM_LOI // 16):
                        obuf[pl.ds(p_loc * DIM_LOI + 16 * j, 16)] = accs[j]
                return 0

            lax.fori_loop(0, G_PTS // 16, chunk, 0)
            obase = (wid * LINES_PER_W + g * G_LINES) * DIM_LOI * N_PTS1
            pltpu.sync_copy(obuf,
                            out_h.at[pl.ds(obase, G_LINES * DIM_LOI * N_PTS1)])

        bufsA = (a00, a10, a01, a11)
        bufsB = (b00, b10, b01, b11)
        fire(0, bufsA, semA)

        def body(gg, _):
            g0 = gg * 2
            cB = [pltpu.async_copy(
                table_h.at[iv.at[pl.ds((g0 + 1) * G_PTS, G_PTS)]], rb, semB)
                for iv, rb in zip(ivs, bufsB)]
            drain(bufsA, semA)
            compute(g0, bufsA)
            fire(jnp.minimum(g0 + 2, N_GROUPS - 1), bufsA, semA)
            for c in cB:
                c.wait()
            compute(g0 + 1, bufsB)
            return 0

        lax.fori_loop(0, N_GROUPS // 2, body, 0)
        drain(bufsA, semA)

    return k(table, i00, i10, i01, i11, w00, w10, w01, w11)


def _mlp_body(x_ref, w1_ref, b1_ref, w2_ref, b2_ref, w3_ref, b3_ref, o_ref):
    dn = (((1,), (0,)), ((), ()))
    h1 = jax.nn.relu(lax.dot_general(x_ref[...], w1_ref[...], dn,
                                     preferred_element_type=jnp.float32)
                     + b1_ref[...])
    h2 = jax.nn.relu(lax.dot_general(h1, w2_ref[...], dn,
                                     preferred_element_type=jnp.float32)
                     + b2_ref[...])
    lg = lax.dot_general(h2, w3_ref[...], dn,
                         preferred_element_type=jnp.float32) + b3_ref[...]
    m = jnp.max(lg, axis=1, keepdims=True)
    e = jnp.exp(lg - m)
    o_ref[...] = e / jnp.sum(e, axis=1, keepdims=True)


def _mlp_call(feat, w1p, b1_row, w2, b2_row, w3p, b3p_row):
    blk = 1024
    grid = NLINES // blk
    return pl.pallas_call(
        _mlp_body,
        grid=(grid,),
        in_specs=[
            pl.BlockSpec((blk, DIM_FC), lambda i: (i, 0)),
            pl.BlockSpec((DIM_FC, DIM_FC), lambda i: (0, 0)),
            pl.BlockSpec((1, DIM_FC), lambda i: (0, 0)),
            pl.BlockSpec((DIM_FC, DIM_FC), lambda i: (0, 0)),
            pl.BlockSpec((1, DIM_FC), lambda i: (0, 0)),
            pl.BlockSpec((DIM_FC, 128), lambda i: (0, 0)),
            pl.BlockSpec((1, 128), lambda i: (0, 0)),
        ],
        out_specs=pl.BlockSpec((blk, 128), lambda i: (i, 0)),
        out_shape=jax.ShapeDtypeStruct((NLINES, 128), jnp.float32),
    )(feat, w1p, b1_row, w2, b2_row, w3p, b3p_row)


def kernel(feature, jmap, joff, fc1_w, fc1_b, w1, b1, w2, b2, w3, b3):
    feat2d = feature.reshape(C_FEAT, NPIX)
    jmap2d = jmap.reshape(H, W)
    joff0 = joff[0, 0, 0]
    joff1 = joff[0, 0, 1]

    table = _fc1_call(feat2d, fc1_w, fc1_b.reshape(1, DIM_LOI))

    i00, i10, i01, i11, w00, w10, w01, w11 = _junction_call(
        jmap2d, joff0, joff1)

    feat_lines = _sc_gather_kernel(
        table,
        i00.reshape(NPTS), i10.reshape(NPTS),
        i01.reshape(NPTS), i11.reshape(NPTS),
        w00.reshape(NPTS), w10.reshape(NPTS),
        w01.reshape(NPTS), w11.reshape(NPTS),
    ).reshape(NLINES, DIM_FC)

    w1p = w1.reshape(DIM_LOI, N_PTS1, DIM_FC).transpose(1, 0, 2).reshape(
        DIM_FC, DIM_FC)
    w3p = jnp.pad(w3, ((0, 0), (0, 125)))
    b3p = jnp.concatenate([b3, jnp.full((125,), -1e30, jnp.float32)])

    probs = _mlp_call(feat_lines, w1p, b1.reshape(1, DIM_FC),
                      w2, b2.reshape(1, DIM_FC), w3p, b3p.reshape(1, 128))
    return probs[:, :3]
